# baseline (speedup 1.0000x reference)
import functools

import jax
import jax.numpy as jnp
from jax import lax
from jax.experimental import pallas as pl
from jax.experimental.pallas import tpu as pltpu
from jax.experimental.pallas import tpu_sc as plsc

_NUM_Q = 8
_K = 1024
_DIM = 256
_NTOK = 16384
_TB = 1024
_SCALE = 1.25 / float(_NTOK * _DIM)


_NW = 32
_BPW = _NTOK // _NW
_CH = 128


def _sc_gather_body(table_hbm, idx_hbm, out_hbm, idx_v, rows_v, sem):
    wid = lax.axis_index("s") * 2 + lax.axis_index("c")
    base = wid * _BPW
    for j in range(_BPW // _CH):
        off = base + j * _CH
        pltpu.sync_copy(idx_hbm.at[pl.ds(off, _CH)], idx_v)
        pltpu.async_copy(table_hbm.at[idx_v], rows_v, sem).wait()
        pltpu.sync_copy(rows_v, out_hbm.at[pl.ds(off, _CH)])


_sc_gather = pl.kernel(
    _sc_gather_body,
    mesh=plsc.VectorSubcoreMesh(core_axis_name="c", subcore_axis_name="s"),
    out_type=jax.ShapeDtypeStruct((_NTOK, _DIM), jnp.float32),
    scratch_types=[
        pltpu.VMEM((_CH,), jnp.int32),
        pltpu.VMEM((_CH, _DIM), jnp.float32),
        pltpu.SemaphoreType.DMA,
    ],
)


def _dist_body_prev(cb_ref, rprev_ref, qprev_ref, idx_ref, rout_ref,
                    loss_ref, cnorm_ref):
    _dist_common(cb_ref, rprev_ref, qprev_ref, idx_ref, rout_ref,
                 loss_ref, cnorm_ref)


def _dist_body_first(cb_ref, rprev_ref, idx_ref, rout_ref,
                     loss_ref, cnorm_ref):
    _dist_common(cb_ref, rprev_ref, None, idx_ref, rout_ref,
                 loss_ref, cnorm_ref)


def _dist_common(cb_ref, rprev_ref, qprev_ref, idx_ref, rout_ref,
                 loss_ref, cnorm_ref):
    @pl.when(pl.program_id(0) == 0)
    def _init():
        cb3 = cb_ref[...]
        cnorm_ref[...] = jnp.sum(cb3 * cb3, axis=-1)[None, :]
        loss_ref[...] = jnp.zeros_like(loss_ref)

    if qprev_ref is None:
        r = rprev_ref[...]
    else:
        r = rprev_ref[...] - qprev_ref[...]
    dots = jax.lax.dot_general(
        r, cb_ref[...], (((1,), (1,)), ((), ())),
        preferred_element_type=jnp.float32,
        precision=jax.lax.Precision.DEFAULT)
    rnorm = jnp.sum(r * r, axis=1, keepdims=True)
    if qprev_ref is not None:
        loss_ref[...] += jnp.broadcast_to(jnp.sum(rnorm) * _SCALE, (1, 128))
    d = rnorm - 2.0 * dots + cnorm_ref[...]
    dmin = jnp.min(d, axis=1, keepdims=True)
    iota = jax.lax.broadcasted_iota(jnp.int32, d.shape, 1)
    idx = jnp.min(jnp.where(d == dmin, iota, _K), axis=1, keepdims=True)
    idx_ref[...] = idx
    rout_ref[...] = r


def _make_dist(first):
    body = _dist_body_first if first else _dist_body_prev
    n_in = 2 if first else 3
    in_specs = [pl.BlockSpec((_K, _DIM), lambda i: (0, 0)),
                pl.BlockSpec((_TB, _DIM), lambda i: (i, 0)),
                pl.BlockSpec((_TB, _DIM), lambda i: (i, 0))][:n_in]
    return pl.pallas_call(
        body,
        grid=(_NTOK // _TB,),
        in_specs=in_specs,
        out_specs=[
            pl.BlockSpec((_TB, 1), lambda i: (i, 0)),
            pl.BlockSpec((_TB, _DIM), lambda i: (i, 0)),
            pl.BlockSpec((1, 128), lambda i: (0, 0)),
        ],
        out_shape=[
            jax.ShapeDtypeStruct((_NTOK, 1), jnp.int32),
            jax.ShapeDtypeStruct((_NTOK, _DIM), jnp.float32),
            jax.ShapeDtypeStruct((1, 128), jnp.float32),
        ],
        scratch_shapes=[pltpu.VMEM((1, _K), jnp.float32)],
    )


_dist_first = _make_dist(True)
_dist_next = _make_dist(False)


def _final_body(x_ref, r_ref, q_ref, qout_ref, loss_ref):
    @pl.when(pl.program_id(0) == 0)
    def _init():
        loss_ref[...] = jnp.zeros_like(loss_ref)

    r8 = r_ref[...] - q_ref[...]
    qout_ref[...] = x_ref[...] - r8
    loss_ref[...] += jnp.broadcast_to(jnp.sum(r8 * r8) * _SCALE, (1, 128))


_final = pl.pallas_call(
    _final_body,
    grid=(_NTOK // _TB,),
    in_specs=[pl.BlockSpec((_TB, _DIM), lambda i: (i, 0))] * 3,
    out_specs=[
        pl.BlockSpec((_TB, _DIM), lambda i: (i, 0)),
        pl.BlockSpec((1, 128), lambda i: (0, 0)),
    ],
    out_shape=[
        jax.ShapeDtypeStruct((_NTOK, _DIM), jnp.float32),
        jax.ShapeDtypeStruct((1, 128), jnp.float32),
    ],
)


def kernel(x, codebooks):
    b, t, dim = x.shape
    x2 = x.reshape(_NTOK, dim)
    r = x2
    quant = None
    idx_cols = []
    loss_vals = []
    for q in range(_NUM_Q):
        cbq = codebooks[q]
        if q == 0:
            idx_q, r, lp = _dist_first(cbq, r)
        else:
            idx_q, r, lp = _dist_next(cbq, r, quant)
            loss_vals.append(lp[0, 0])
        idx_cols.append(idx_q)
        quant = _sc_gather(cbq, idx_q.reshape(_NTOK))
    qout2, lp = _final(x2, r, quant)
    loss_vals.append(lp[0, 0])
    quantized = qout2.reshape(b, t, dim)
    indices = jnp.concatenate(idx_cols, axis=1).T.reshape(_NUM_Q, b, t)
    losses = jnp.stack(loss_vals)
    return quantized, indices, losses


# speedup vs baseline: 1.2539x; 1.2539x over previous
import jax
import jax.numpy as jnp
from jax.experimental import pallas as pl
from jax.experimental.pallas import tpu as pltpu

_NUM_Q = 8
_K = 1024
_DIM = 256
_TB = 1024
_SPLIT = 2


def _rvq_body(cb_ref, x_ref, qout_ref, idx_ref, loss_ref, cnorm_ref,
              cbh_ref, cbm_ref, cbl_ref):
    @pl.when(pl.program_id(0) == 0)
    def _init():
        cb3 = cb_ref[...]
        cnorm_ref[...] = jnp.sum(cb3 * cb3, axis=-1)
        loss_ref[...] = jnp.zeros_like(loss_ref)
        hi = cb3.astype(jnp.bfloat16).astype(jnp.float32)
        r1 = cb3 - hi
        mid = r1.astype(jnp.bfloat16).astype(jnp.float32)
        lo = (r1 - mid).astype(jnp.bfloat16).astype(jnp.float32)
        cbh_ref[...] = hi
        cbm_ref[...] = mid
        cbl_ref[...] = lo

    def layer_step(q, residual, qout):
        cb = cb_ref[q]
        dots = jax.lax.dot_general(
            residual, cb, (((1,), (1,)), ((), ())),
            preferred_element_type=jnp.float32,
            precision=jax.lax.Precision.DEFAULT)
        rnorm = jnp.sum(residual * residual, axis=1, keepdims=True)
        d = rnorm - 2.0 * dots + cnorm_ref[q:q + 1, :]
        idx = jnp.argmin(d, axis=1)[:, None]
        iota = jax.lax.broadcasted_iota(jnp.int32, d.shape, 1)
        onehot = (iota == idx).astype(jnp.float32)
        dn = (((1,), (0,)), ((), ()))
        quant = (jax.lax.dot_general(
                     onehot, cbh_ref[q], dn,
                     preferred_element_type=jnp.float32,
                     precision=jax.lax.Precision.DEFAULT)
                 + jax.lax.dot_general(
                     onehot, cbm_ref[q], dn,
                     preferred_element_type=jnp.float32,
                     precision=jax.lax.Precision.DEFAULT)
                 + jax.lax.dot_general(
                     onehot, cbl_ref[q], dn,
                     preferred_element_type=jnp.float32,
                     precision=jax.lax.Precision.DEFAULT))
        return residual - quant, qout + quant, idx, jnp.sum(rnorm)

    h = _TB // _SPLIT
    res = [x_ref[s * h:(s + 1) * h, :] for s in range(_SPLIT)]
    qo = [jnp.zeros((h, _DIM), jnp.float32) for _ in range(_SPLIT)]
    idx_cols = [[] for _ in range(_SPLIT)]
    loss_parts = [[] for _ in range(_SPLIT)]
    for q in range(_NUM_Q):
        for s in range(_SPLIT):
            res[s], qo[s], idx, rn = layer_step(q, res[s], qo[s])
            idx_cols[s].append(idx)
            if q > 0:
                loss_parts[s].append(rn)
    for s in range(_SPLIT):
        loss_parts[s].append(jnp.sum(res[s] * res[s]))
        qout_ref[s * h:(s + 1) * h, :] = qo[s]
        idx_ref[s * h:(s + 1) * h, :] = jnp.concatenate(idx_cols[s], axis=1)
    scale = 1.25 / float(16 * 1024 * _DIM)
    totals = [sum(parts[q] for parts in loss_parts) * scale
              for q in range(_NUM_Q)]
    loss_ref[...] += jnp.stack(
        [jnp.broadcast_to(t, (128,)) for t in totals])


def kernel(x, codebooks):
    b, t, dim = x.shape
    ntok = b * t
    x2 = x.reshape(ntok, dim)
    qout2, idx_t, loss_mat = pl.pallas_call(
        _rvq_body,
        grid=(ntok // _TB,),
        in_specs=[
            pl.BlockSpec((_NUM_Q, _K, _DIM), lambda i: (0, 0, 0)),
            pl.BlockSpec((_TB, _DIM), lambda i: (i, 0)),
        ],
        out_specs=[
            pl.BlockSpec((_TB, _DIM), lambda i: (i, 0)),
            pl.BlockSpec((_TB, _NUM_Q), lambda i: (i, 0)),
            pl.BlockSpec((_NUM_Q, 128), lambda i: (0, 0)),
        ],
        out_shape=[
            jax.ShapeDtypeStruct((ntok, dim), jnp.float32),
            jax.ShapeDtypeStruct((ntok, _NUM_Q), jnp.int32),
            jax.ShapeDtypeStruct((_NUM_Q, 128), jnp.float32),
        ],
        scratch_shapes=[
            pltpu.VMEM((_NUM_Q, _K), jnp.float32),
            pltpu.VMEM((_NUM_Q, _K, _DIM), jnp.float32),
            pltpu.VMEM((_NUM_Q, _K, _DIM), jnp.float32),
            pltpu.VMEM((_NUM_Q, _K, _DIM), jnp.float32),
        ],
    )(codebooks, x2)
    quantized = qout2.reshape(b, t, dim)
    indices = idx_t.T.reshape(_NUM_Q, b, t)
    losses = loss_mat[:, 0]
    return quantized, indices, losses


# speedup vs baseline: 1.3060x; 1.0415x over previous
import jax
import jax.numpy as jnp
from jax.experimental import pallas as pl
from jax.experimental.pallas import tpu as pltpu

_NUM_Q = 8
_K = 1024
_DIM = 256
_TB = 1024
_SPLIT = 2


def _rvq_body(cb_ref, x_ref, qout_ref, idx_ref, loss_ref, cnorm_ref,
              cbh_ref, cbm_ref, cbl_ref):
    @pl.when(pl.program_id(0) == 0)
    def _init():
        cb3 = cb_ref[...]
        cnorm_ref[...] = jnp.sum(cb3 * cb3, axis=-1)
        loss_ref[...] = jnp.zeros_like(loss_ref)
        hi = cb3.astype(jnp.bfloat16).astype(jnp.float32)
        r1 = cb3 - hi
        mid = r1.astype(jnp.bfloat16).astype(jnp.float32)
        lo = (r1 - mid).astype(jnp.bfloat16).astype(jnp.float32)
        cbh_ref[...] = hi
        cbm_ref[...] = mid
        cbl_ref[...] = lo

    def layer_step(q, residual, qout):
        cb = cb_ref[q]
        dots = jax.lax.dot_general(
            residual, cb, (((1,), (1,)), ((), ())),
            preferred_element_type=jnp.float32,
            precision=jax.lax.Precision.DEFAULT)
        rnorm = jnp.sum(residual * residual, axis=1, keepdims=True)
        d = rnorm - 2.0 * dots + cnorm_ref[q:q + 1, :]
        dmin = jnp.min(d, axis=1, keepdims=True)
        iota = jax.lax.broadcasted_iota(jnp.int32, d.shape, 1)
        idx = jnp.min(jnp.where(d == dmin, iota, _K), axis=1,
                      keepdims=True)
        onehot = (iota == idx).astype(jnp.float32)
        dn = (((1,), (0,)), ((), ()))
        quant = (jax.lax.dot_general(
                     onehot, cbh_ref[q], dn,
                     preferred_element_type=jnp.float32,
                     precision=jax.lax.Precision.DEFAULT)
                 + jax.lax.dot_general(
                     onehot, cbm_ref[q], dn,
                     preferred_element_type=jnp.float32,
                     precision=jax.lax.Precision.DEFAULT)
                 + jax.lax.dot_general(
                     onehot, cbl_ref[q], dn,
                     preferred_element_type=jnp.float32,
                     precision=jax.lax.Precision.DEFAULT))
        return residual - quant, qout + quant, idx, jnp.sum(rnorm)

    h = _TB // _SPLIT
    res = [x_ref[s * h:(s + 1) * h, :] for s in range(_SPLIT)]
    qo = [jnp.zeros((h, _DIM), jnp.float32) for _ in range(_SPLIT)]
    idx_cols = [[] for _ in range(_SPLIT)]
    loss_parts = [[] for _ in range(_SPLIT)]
    for q in range(_NUM_Q):
        for s in range(_SPLIT):
            res[s], qo[s], idx, rn = layer_step(q, res[s], qo[s])
            idx_cols[s].append(idx)
            if q > 0:
                loss_parts[s].append(rn)
    for s in range(_SPLIT):
        loss_parts[s].append(jnp.sum(res[s] * res[s]))
        qout_ref[s * h:(s + 1) * h, :] = qo[s]
        idx_ref[s * h:(s + 1) * h, :] = jnp.concatenate(idx_cols[s], axis=1)
    scale = 1.25 / float(16 * 1024 * _DIM)
    totals = [sum(parts[q] for parts in loss_parts) * scale
              for q in range(_NUM_Q)]
    loss_ref[...] += jnp.stack(
        [jnp.broadcast_to(t, (128,)) for t in totals])


def kernel(x, codebooks):
    b, t, dim = x.shape
    ntok = b * t
    x2 = x.reshape(ntok, dim)
    qout2, idx_t, loss_mat = pl.pallas_call(
        _rvq_body,
        grid=(ntok // _TB,),
        in_specs=[
            pl.BlockSpec((_NUM_Q, _K, _DIM), lambda i: (0, 0, 0)),
            pl.BlockSpec((_TB, _DIM), lambda i: (i, 0)),
        ],
        out_specs=[
            pl.BlockSpec((_TB, _DIM), lambda i: (i, 0)),
            pl.BlockSpec((_TB, _NUM_Q), lambda i: (i, 0)),
            pl.BlockSpec((_NUM_Q, 128), lambda i: (0, 0)),
        ],
        out_shape=[
            jax.ShapeDtypeStruct((ntok, dim), jnp.float32),
            jax.ShapeDtypeStruct((ntok, _NUM_Q), jnp.int32),
            jax.ShapeDtypeStruct((_NUM_Q, 128), jnp.float32),
        ],
        scratch_shapes=[
            pltpu.VMEM((_NUM_Q, _K), jnp.float32),
            pltpu.VMEM((_NUM_Q, _K, _DIM), jnp.float32),
            pltpu.VMEM((_NUM_Q, _K, _DIM), jnp.float32),
            pltpu.VMEM((_NUM_Q, _K, _DIM), jnp.float32),
        ],
    )(codebooks, x2)
    quantized = qout2.reshape(b, t, dim)
    indices = idx_t.T.reshape(_NUM_Q, b, t)
    losses = loss_mat[:, 0]
    return quantized, indices, losses
